# baseline (speedup 1.0000x reference)
import jax
import jax.numpy as jnp
from jax import lax
from jax.experimental import pallas as pl
from jax.experimental.pallas import tpu as pltpu
from jax.experimental.pallas import tpu_sc as plsc

_F = 250
_NB = 20
_N = _F * _NB
_BR = 8
_NBANDS = _N // _BR
_BPW = 20
_NW = 32
_NCHUNK = (_BR * _NB) // 16
_CPR = _N // 16
_TROWS = _BR * _BPW
_TPAD = _NW * _TROWS


def _iou16(atab, btab, ai, bi, c0, c1, c2, c3):
    ax1 = plsc.load_gather(atab, [ai, c0])
    ay1 = plsc.load_gather(atab, [ai, c1])
    ax2 = plsc.load_gather(atab, [ai, c2])
    ay2 = plsc.load_gather(atab, [ai, c3])
    bx1 = plsc.load_gather(btab, [bi, c0])
    by1 = plsc.load_gather(btab, [bi, c1])
    bx2 = plsc.load_gather(btab, [bi, c2])
    by2 = plsc.load_gather(btab, [bi, c3])

    inter_x1 = jnp.maximum(ax1, bx1)
    inter_x2 = jnp.minimum(ax2, bx2)
    inter_y1 = jnp.maximum(ay1, by1)
    inter_y2 = jnp.minimum(ay2, by2)
    inter_area = (
        jnp.maximum(inter_x2 - inter_x1, 0.0)
        * jnp.maximum(inter_y2 - inter_y1, 0.0)
    )
    boxa_area = (ax2 - ax1 + 1.0) * (ay2 - ay1 + 1.0)
    boxb_area = (bx2 - bx1 + 1.0) * (bx2 - by1 + 1.0)
    return inter_area / (boxa_area + boxb_area - inter_area)


def _sc_body(a_hbm, b_hbm, o_hbm, zbuf0, zbuf1, atab, btab, sem0, sem1):
    w = lax.axis_index("s") * 2 + lax.axis_index("c")
    iota = lax.iota(jnp.int32, 16)
    zeros16 = jnp.zeros((16,), jnp.float32)
    c0 = jnp.full((16,), 0, jnp.int32)
    c1 = jnp.full((16,), 1, jnp.int32)
    c2 = jnp.full((16,), 2, jnp.int32)
    c3 = jnp.full((16,), 3, jnp.int32)
    tbase = _TROWS * w

    def dyn(n):
        return jnp.where(w >= 0, n, 0)

    pltpu.sync_copy(a_hbm.at[pl.ds(tbase, _TROWS), :], atab)
    pltpu.sync_copy(b_hbm.at[pl.ds(tbase, _TROWS), :], btab)

    tail_mask = iota < (_N - _CPR * 16)
    tail_col = _CPR * 16 + iota
    for zb in (zbuf0, zbuf1):
        def zrow(r, carry, zb=zb):
            for ci in range(_CPR):
                zb[r, pl.ds(16 * ci, 16)] = zeros16
            rv = jnp.broadcast_to(r, (16,))
            plsc.store_scatter(zb, [rv, tail_col], zeros16, mask=tail_mask)
            return carry
        lax.fori_loop(0, dyn(_BR), zrow, 0)

    def window_idx(u, cc):
        f = 16 * cc + iota
        i_c = f // _NB
        j = f % _NB
        gr = _BR * u + i_c
        blk = gr // _NB
        col = _NB * blk + j
        return i_c, j, blk, col

    def zero_window(zb, u):
        def chunk(cc, carry):
            i_c, _, _, col = window_idx(u, cc)
            plsc.store_scatter(zb, [i_c, col], zeros16)
            return carry
        lax.fori_loop(0, dyn(_NCHUNK), chunk, 0)

    def fill_window(zb, u, m):
        def chunk(cc, carry):
            i_c, j, blk, col = window_idx(u, cc)
            ai = _BR * m + i_c
            bi = col - tbase
            iou = _iou16(atab, btab, ai, bi, c0, c1, c2, c3)
            val = jnp.where(blk != 248, iou, 0.0)
            plsc.store_scatter(zb, [i_c, col], val)
            return carry
        lax.fori_loop(0, dyn(_NCHUNK), chunk, 0)

    def drain(zb, sem):
        pltpu.make_async_copy(
            zb, o_hbm.at[pl.ds(0, _BR), :], sem
        ).wait()

    def band(p, m, zb, sem):
        u = _BPW * w + m

        @pl.when(u < _NBANDS)
        def _do():
            @pl.when(p >= 1)
            def _recycle():
                drain(zb, sem)
                zero_window(zb, u - 2)

            fill_window(zb, u, m)
            pltpu.make_async_copy(
                zb,
                o_hbm.at[pl.ds(_BR * u, _BR), :],
                sem,
            ).start()

    def pair(p, carry):
        band(p, 2 * p, zbuf0, sem0)
        band(p, 2 * p + 1, zbuf1, sem1)
        return carry

    lax.fori_loop(0, dyn(_BPW // 2), pair, 0)

    nvalid = jnp.clip(_NBANDS - _BPW * w, 0, _BPW)

    @pl.when(nvalid > 0)
    def _drain0():
        drain(zbuf0, sem0)

    @pl.when(nvalid > 1)
    def _drain1():
        drain(zbuf1, sem1)


def kernel(rois):
    a_tbl = jnp.roll(rois, -1, axis=0).reshape(_N, 4)
    b_tbl = jnp.roll(rois, -2, axis=0).reshape(_N, 4)
    a_tbl = jnp.pad(a_tbl, ((0, _TPAD - _N), (0, 0)))
    b_tbl = jnp.pad(b_tbl, ((0, _TPAD - _N), (0, 0)))

    mesh = plsc.VectorSubcoreMesh(core_axis_name="c", subcore_axis_name="s")
    sc = pl.kernel(
        _sc_body,
        out_type=jax.ShapeDtypeStruct((_N, _N), jnp.float32),
        mesh=mesh,
        scratch_types=[
            pltpu.VMEM((_BR, _N), jnp.float32),
            pltpu.VMEM((_BR, _N), jnp.float32),
            pltpu.VMEM((_TROWS, 4), jnp.float32),
            pltpu.VMEM((_TROWS, 4), jnp.float32),
            pltpu.SemaphoreType.DMA,
            pltpu.SemaphoreType.DMA,
        ],
        compiler_params=pltpu.CompilerParams(
            use_tc_tiling_on_sc=False, needs_layout_passes=False
        ),
    )
    out = sc(a_tbl, b_tbl)
    return out.reshape(1, _N, _N)
